# Initial kernel scaffold; baseline (speedup 1.0000x reference)
import jax
import jax.numpy as jnp
from jax.experimental import pallas as pl


def kernel(features, negative_features, adj, W_gcn, b_gcn, prelu_alpha, disc_W, disc_b):
    raise NotImplementedError("write your pallas kernel here")


# speedup vs baseline: 1.6392x; 1.6392x over previous
import jax
import jax.numpy as jnp
from jax.experimental import pallas as pl


def _seq_kernel(f_ref, n_ref, w_ref, s_ref):
    w = w_ref[:]
    F = w.shape[1]
    s_ref[:, :F] = jnp.dot(f_ref[:], w, preferred_element_type=jnp.float32)
    s_ref[:, F:] = jnp.dot(n_ref[:], w, preferred_element_type=jnp.float32)


def _gcn_kernel(adj_ref, s_ref, b2_ref, alpha_ref, h_ref):
    acc = jnp.dot(adj_ref[:], s_ref[:], preferred_element_type=jnp.float32)
    h = acc + b2_ref[:]
    alpha = alpha_ref[0, 0]
    h_ref[:] = jnp.where(h >= 0, h, alpha * h)


def _disc_kernel(h_ref, dwt_ref, db_ref, sc1_ref, sc2_ref):
    n = h_ref.shape[0]
    F = dwt_ref.shape[0]
    hp = h_ref[:, :F]
    hn = h_ref[:, F:]
    c = jnp.sum(hp, axis=0, keepdims=True) * (1.0 / n)
    s = jax.nn.sigmoid(c)
    u = jnp.dot(s, dwt_ref[:], preferred_element_type=jnp.float32)
    db = db_ref[0, 0]
    sc1_ref[:] = jnp.sum(hp * u, axis=1, keepdims=True) + db
    sc2_ref[:] = jnp.sum(hn * u, axis=1, keepdims=True) + db


def kernel(features, negative_features, adj, W_gcn, b_gcn, prelu_alpha, disc_W, disc_b):
    B, N, IN_F = features.shape
    OUT_F = W_gcn.shape[1]
    f2 = features.reshape(N, IN_F)
    n2 = negative_features.reshape(N, IN_F)
    adj2 = adj.reshape(N, N)
    b2 = jnp.concatenate([b_gcn, b_gcn]).reshape(1, 2 * OUT_F)
    alpha = prelu_alpha.reshape(1, 1)
    db = disc_b.reshape(1, 1)
    dwt = disc_W.T

    S = pl.pallas_call(
        _seq_kernel,
        out_shape=jax.ShapeDtypeStruct((N, 2 * OUT_F), jnp.float32),
    )(f2, n2, W_gcn)

    BM = 400
    h = pl.pallas_call(
        _gcn_kernel,
        grid=(N // BM,),
        in_specs=[
            pl.BlockSpec((BM, N), lambda i: (i, 0)),
            pl.BlockSpec((N, 2 * OUT_F), lambda i: (0, 0)),
            pl.BlockSpec((1, 2 * OUT_F), lambda i: (0, 0)),
            pl.BlockSpec((1, 1), lambda i: (0, 0)),
        ],
        out_specs=pl.BlockSpec((BM, 2 * OUT_F), lambda i: (i, 0)),
        out_shape=jax.ShapeDtypeStruct((N, 2 * OUT_F), jnp.float32),
    )(adj2, S, b2, alpha)

    sc1, sc2 = pl.pallas_call(
        _disc_kernel,
        out_shape=[
            jax.ShapeDtypeStruct((N, 1), jnp.float32),
            jax.ShapeDtypeStruct((N, 1), jnp.float32),
        ],
    )(h, dwt, db)

    return jnp.concatenate([sc1.reshape(1, N), sc2.reshape(1, N)], axis=1)


# speedup vs baseline: 1.6539x; 1.0090x over previous
import jax
import jax.numpy as jnp
from jax.experimental import pallas as pl


def _seq_kernel(f_ref, n_ref, w_ref, s_ref):
    w = w_ref[:]
    F = w.shape[1]
    s_ref[:, :F] = jnp.dot(
        f_ref[:], w, preferred_element_type=jnp.float32
    ).astype(jnp.bfloat16)
    s_ref[:, F:] = jnp.dot(
        n_ref[:], w, preferred_element_type=jnp.float32
    ).astype(jnp.bfloat16)


def _gcn_kernel(adj_ref, s_ref, b2_ref, alpha_ref, h_ref):
    acc = jnp.dot(adj_ref[:].astype(jnp.bfloat16), s_ref[:],
                  preferred_element_type=jnp.float32)
    h = acc + b2_ref[:]
    alpha = alpha_ref[0, 0]
    h_ref[:] = jnp.where(h >= 0, h, alpha * h)


def _disc_kernel(h_ref, dwt_ref, db_ref, sc1_ref, sc2_ref):
    n = h_ref.shape[0]
    F = dwt_ref.shape[0]
    hp = h_ref[:, :F]
    hn = h_ref[:, F:]
    c = jnp.sum(hp, axis=0, keepdims=True) * (1.0 / n)
    s = jax.nn.sigmoid(c)
    u = jnp.dot(s, dwt_ref[:], preferred_element_type=jnp.float32)
    db = db_ref[0, 0]
    sc1_ref[:] = jnp.sum(hp * u, axis=1, keepdims=True) + db
    sc2_ref[:] = jnp.sum(hn * u, axis=1, keepdims=True) + db


def kernel(features, negative_features, adj, W_gcn, b_gcn, prelu_alpha, disc_W, disc_b):
    B, N, IN_F = features.shape
    OUT_F = W_gcn.shape[1]
    f2 = features.reshape(N, IN_F)
    n2 = negative_features.reshape(N, IN_F)
    adj2 = adj.reshape(N, N)
    b2 = jnp.concatenate([b_gcn, b_gcn]).reshape(1, 2 * OUT_F)
    alpha = prelu_alpha.reshape(1, 1)
    db = disc_b.reshape(1, 1)
    dwt = disc_W.T

    S = pl.pallas_call(
        _seq_kernel,
        out_shape=jax.ShapeDtypeStruct((N, 2 * OUT_F), jnp.bfloat16),
    )(f2, n2, W_gcn)

    BM = 400
    h = pl.pallas_call(
        _gcn_kernel,
        grid=(N // BM,),
        in_specs=[
            pl.BlockSpec((BM, N), lambda i: (i, 0)),
            pl.BlockSpec((N, 2 * OUT_F), lambda i: (0, 0)),
            pl.BlockSpec((1, 2 * OUT_F), lambda i: (0, 0)),
            pl.BlockSpec((1, 1), lambda i: (0, 0)),
        ],
        out_specs=pl.BlockSpec((BM, 2 * OUT_F), lambda i: (i, 0)),
        out_shape=jax.ShapeDtypeStruct((N, 2 * OUT_F), jnp.float32),
    )(adj2, S, b2, alpha)

    sc1, sc2 = pl.pallas_call(
        _disc_kernel,
        out_shape=[
            jax.ShapeDtypeStruct((N, 1), jnp.float32),
            jax.ShapeDtypeStruct((N, 1), jnp.float32),
        ],
    )(h, dwt, db)

    return jnp.concatenate([sc1.reshape(1, N), sc2.reshape(1, N)], axis=1)


# speedup vs baseline: 1.7458x; 1.0555x over previous
import jax
import jax.numpy as jnp
from jax.experimental import pallas as pl
from jax.experimental.pallas import tpu as pltpu


def _gcn_kernel(adj_ref, f_ref, n_ref, w_ref, b2_ref, alpha_ref, h_ref, s_ref):
    F = w_ref.shape[1]

    @pl.when(pl.program_id(0) == 0)
    def _():
        w = w_ref[:]
        s_ref[:, :F] = jnp.dot(
            f_ref[:], w, preferred_element_type=jnp.float32
        ).astype(jnp.bfloat16)
        s_ref[:, F:] = jnp.dot(
            n_ref[:], w, preferred_element_type=jnp.float32
        ).astype(jnp.bfloat16)

    acc = jnp.dot(adj_ref[:].astype(jnp.bfloat16), s_ref[:],
                  preferred_element_type=jnp.float32)
    h = acc + b2_ref[:]
    alpha = alpha_ref[0, 0]
    h_ref[:] = jnp.where(h >= 0, h, alpha * h).astype(jnp.bfloat16)


def _disc_kernel(h_ref, dwt_ref, db_ref, sc1_ref, sc2_ref):
    n = h_ref.shape[0]
    F = dwt_ref.shape[0]
    hp = h_ref[:, :F].astype(jnp.float32)
    hn = h_ref[:, F:].astype(jnp.float32)
    c = jnp.sum(hp, axis=0, keepdims=True) * (1.0 / n)
    s = jax.nn.sigmoid(c)
    u = jnp.dot(s, dwt_ref[:], preferred_element_type=jnp.float32)
    db = db_ref[0, 0]
    sc1_ref[:] = jnp.sum(hp * u, axis=1, keepdims=True) + db
    sc2_ref[:] = jnp.sum(hn * u, axis=1, keepdims=True) + db


def kernel(features, negative_features, adj, W_gcn, b_gcn, prelu_alpha, disc_W, disc_b):
    B, N, IN_F = features.shape
    OUT_F = W_gcn.shape[1]
    f2 = features.reshape(N, IN_F)
    n2 = negative_features.reshape(N, IN_F)
    adj2 = adj.reshape(N, N)
    b2 = jnp.concatenate([b_gcn, b_gcn]).reshape(1, 2 * OUT_F)
    alpha = prelu_alpha.reshape(1, 1)
    db = disc_b.reshape(1, 1)
    dwt = disc_W.T

    BM = 400
    h = pl.pallas_call(
        _gcn_kernel,
        grid=(N // BM,),
        in_specs=[
            pl.BlockSpec((BM, N), lambda i: (i, 0)),
            pl.BlockSpec((N, IN_F), lambda i: (0, 0)),
            pl.BlockSpec((N, IN_F), lambda i: (0, 0)),
            pl.BlockSpec((IN_F, OUT_F), lambda i: (0, 0)),
            pl.BlockSpec((1, 2 * OUT_F), lambda i: (0, 0)),
            pl.BlockSpec((1, 1), lambda i: (0, 0)),
        ],
        out_specs=pl.BlockSpec((BM, 2 * OUT_F), lambda i: (i, 0)),
        out_shape=jax.ShapeDtypeStruct((N, 2 * OUT_F), jnp.bfloat16),
        scratch_shapes=[pltpu.VMEM((N, 2 * OUT_F), jnp.bfloat16)],
    )(adj2, f2, n2, W_gcn, b2, alpha)

    sc1, sc2 = pl.pallas_call(
        _disc_kernel,
        out_shape=[
            jax.ShapeDtypeStruct((N, 1), jnp.float32),
            jax.ShapeDtypeStruct((N, 1), jnp.float32),
        ],
    )(h, dwt, db)

    return jnp.concatenate([sc1.reshape(1, N), sc2.reshape(1, N)], axis=1)
